# Initial kernel scaffold; baseline (speedup 1.0000x reference)
import functools

import jax
import jax.numpy as jnp
from jax import lax
from jax.experimental import pallas as pl
from jax.experimental.pallas import tpu as pltpu
from jax.experimental.pallas import tpu_sc as plsc

VOCAB = 50257
D = 64
B = 16384
L = 200
NUM_CLASSES = 2

NC = 2
NS = 16
NW = NC * NS
B_PER_W = B // NW
CHUNK = 4
NCHUNKS = B_PER_W // CHUNK


def _sc_pool(input_ids, emb):
  mesh = plsc.VectorSubcoreMesh(core_axis_name="c", subcore_axis_name="s")

  @functools.partial(
      pl.kernel,
      mesh=mesh,
      out_type=jax.ShapeDtypeStruct((B, D), jnp.float32),
      scratch_types=[
          pltpu.VMEM((CHUNK, L), jnp.int32),
          pltpu.VMEM((CHUNK, L, D), jnp.float32),
          pltpu.VMEM((B_PER_W, D), jnp.float32),
          pltpu.SemaphoreType.DMA,
      ],
  )
  def k(ids_hbm, table_hbm, out_hbm, idx_v, rows_v, stage_v, sem):
    wid = lax.axis_index("s") * NC + lax.axis_index("c")
    base = wid * B_PER_W

    def chunk_body(ci, carry):
      row0 = base + ci * CHUNK
      pltpu.sync_copy(ids_hbm.at[pl.ds(row0, CHUNK)], idx_v)
      copies = [
          pltpu.async_copy(table_hbm.at[idx_v.at[r]], rows_v.at[r], sem)
          for r in range(CHUNK)
      ]
      for cp in copies:
        cp.wait()
      for r in range(CHUNK):
        def red(j, accs, r=r):
          return tuple(
              accs[d] + rows_v[r, j, pl.ds(d * 16, 16)] for d in range(4)
          )
        accs = lax.fori_loop(
            0, L, red,
            tuple(jnp.zeros((16,), jnp.float32) for _ in range(4)))
        out_row = ci * CHUNK + r
        for d in range(4):
          stage_v[out_row, pl.ds(d * 16, 16)] = accs[d]
      return carry

    lax.fori_loop(0, NCHUNKS, chunk_body, 0)
    pltpu.sync_copy(stage_v, out_hbm.at[pl.ds(base, B_PER_W)])

  return k(input_ids, emb)


def _tc_head(pooled, gamma, beta, W, b):
  def body(x_ref, g_ref, be_ref, w_ref, b_ref, o_ref):
    x = x_ref[...] * (1.0 / L)
    mu = jnp.mean(x, axis=-1, keepdims=True)
    xc = x - mu
    var = jnp.mean(xc * xc, axis=-1, keepdims=True)
    y = xc * lax.rsqrt(var + 1e-5) * g_ref[...] + be_ref[...]
    o_ref[...] = (
        lax.dot_general(y, w_ref[...], (((1,), (1,)), ((), ())),
                        preferred_element_type=jnp.float32)
        + b_ref[...]
    )

  return pl.pallas_call(
      body,
      out_shape=jax.ShapeDtypeStruct((B, NUM_CLASSES), jnp.float32),
  )(pooled, gamma.reshape(1, D), beta.reshape(1, D), W,
    b.reshape(1, NUM_CLASSES))


def kernel(input_ids, emb, gamma, beta, W, b):
  ids = input_ids.astype(jnp.int32)
  pooled = _sc_pool(ids, emb)
  return _tc_head(pooled, gamma, beta, W, b)


# speedup vs baseline: 16.1587x; 16.1587x over previous
import functools

import jax
import jax.numpy as jnp
from jax import lax
from jax.experimental import pallas as pl
from jax.experimental.pallas import tpu as pltpu
from jax.experimental.pallas import tpu_sc as plsc

VOCAB = 50257
D = 64
B = 16384
L = 200
NUM_CLASSES = 2

NC = 2
NS = 16
NW = NC * NS
B_PER_W = B // NW
CHUNK = 4
NCHUNKS = B_PER_W // CHUNK


def _sc_pool(input_ids, emb):
  mesh = plsc.VectorSubcoreMesh(core_axis_name="c", subcore_axis_name="s")

  @functools.partial(
      pl.kernel,
      mesh=mesh,
      out_type=jax.ShapeDtypeStruct((B, D), jnp.float32),
      scratch_types=[
          pltpu.VMEM((CHUNK, L), jnp.int32),
          pltpu.VMEM((CHUNK, L, D), jnp.float32),
          pltpu.VMEM((B_PER_W, D), jnp.float32),
          pltpu.SemaphoreType.DMA,
      ],
      compiler_params=pltpu.CompilerParams(use_tc_tiling_on_sc=False),
  )
  def k(ids_hbm, table_hbm, out_hbm, idx_v, rows_v, stage_v, sem):
    wid = lax.axis_index("s") * NC + lax.axis_index("c")
    base = wid * B_PER_W

    def chunk_body(ci, carry):
      row0 = base + ci * CHUNK
      pltpu.sync_copy(ids_hbm.at[pl.ds(row0, CHUNK)], idx_v)
      copies = [
          pltpu.async_copy(table_hbm.at[idx_v.at[r]], rows_v.at[r], sem)
          for r in range(CHUNK)
      ]
      for cp in copies:
        cp.wait()
      for r in range(CHUNK):
        def red(j, accs, r=r):
          return tuple(
              accs[d] + rows_v[r, j, pl.ds(d * 16, 16)] for d in range(4)
          )
        accs = lax.fori_loop(
            0, L, red,
            tuple(jnp.zeros((16,), jnp.float32) for _ in range(4)))
        out_row = ci * CHUNK + r
        for d in range(4):
          stage_v[out_row, pl.ds(d * 16, 16)] = accs[d]
      return carry

    lax.fori_loop(0, NCHUNKS, chunk_body, 0)
    pltpu.sync_copy(stage_v, out_hbm.at[pl.ds(base, B_PER_W)])

  return k(input_ids, emb)


def _tc_head(pooled, gamma, beta, W, b):
  def body(x_ref, g_ref, be_ref, w_ref, b_ref, o_ref):
    x = x_ref[...] * (1.0 / L)
    mu = jnp.mean(x, axis=-1, keepdims=True)
    xc = x - mu
    var = jnp.mean(xc * xc, axis=-1, keepdims=True)
    y = xc * lax.rsqrt(var + 1e-5) * g_ref[...] + be_ref[...]
    o_ref[...] = (
        lax.dot_general(y, w_ref[...], (((1,), (1,)), ((), ())),
                        preferred_element_type=jnp.float32)
        + b_ref[...]
    )

  return pl.pallas_call(
      body,
      out_shape=jax.ShapeDtypeStruct((B, NUM_CLASSES), jnp.float32),
  )(pooled, gamma.reshape(1, D), beta.reshape(1, D), W,
    b.reshape(1, NUM_CLASSES))


def kernel(input_ids, emb, gamma, beta, W, b):
  ids = input_ids.astype(jnp.int32)
  pooled = _sc_pool(ids, emb)
  return _tc_head(pooled, gamma, beta, W, b)


# speedup vs baseline: 25.7427x; 1.5931x over previous
import functools

import jax
import jax.numpy as jnp
from jax import lax
from jax.experimental import pallas as pl
from jax.experimental.pallas import tpu as pltpu
from jax.experimental.pallas import tpu_sc as plsc

VOCAB = 50257
D = 64
B = 16384
L = 200
NUM_CLASSES = 2

NC = 2
NS = 16
NW = NC * NS
B_PER_W = B // NW
CHUNK = 2
JPC = CHUNK * L
NCH = B_PER_W // CHUNK
UNROLL = 8


def _sc_pool(ids_flat, emb):
  mesh = plsc.VectorSubcoreMesh(core_axis_name="c", subcore_axis_name="s")

  @functools.partial(
      pl.kernel,
      mesh=mesh,
      out_type=jax.ShapeDtypeStruct((B, D), jnp.float32),
      scratch_types=[
          pltpu.VMEM((2, JPC), jnp.int32),
          pltpu.VMEM((2, JPC, D), jnp.float32),
          pltpu.VMEM((B_PER_W, D), jnp.float32),
          pltpu.SemaphoreType.DMA,
          pltpu.SemaphoreType.DMA,
          pltpu.SemaphoreType.DMA,
          pltpu.SemaphoreType.DMA,
      ],
      compiler_params=pltpu.CompilerParams(use_tc_tiling_on_sc=False),
  )
  def k(ids_hbm, table_hbm, out_hbm, idx_v, rows_v, stage_v,
        sem_i0, sem_i1, sem_r0, sem_r1):
    sem_i = (sem_i0, sem_i1)
    sem_r = (sem_r0, sem_r1)
    wid = lax.axis_index("s") * NC + lax.axis_index("c")
    ibase = wid * (B_PER_W * L)

    def idx_cp(ci, b):
      return pltpu.make_async_copy(
          ids_hbm.at[pl.ds(ibase + ci * JPC, JPC)], idx_v.at[b], sem_i[b])

    def row_cp(b):
      return pltpu.make_async_copy(
          table_hbm.at[idx_v.at[b]], rows_v.at[b], sem_r[b])

    def reduce_chunk(ci, b):
      for r in range(CHUNK):
        def red(jj, accs, r=r, b=b):
          new = list(accs)
          for u in range(UNROLL):
            j = jj * UNROLL + u
            for d in range(4):
              c = (u % 2) * 4 + d
              new[c] = new[c] + rows_v[b, r * L + j, pl.ds(d * 16, 16)]
          return tuple(new)

        accs = lax.fori_loop(
            0, L // UNROLL, red,
            tuple(jnp.zeros((16,), jnp.float32) for _ in range(8)))
        out_row = ci * CHUNK + r
        for d in range(4):
          stage_v[out_row, pl.ds(d * 16, 16)] = accs[d] + accs[4 + d]

    idx_cp(0, 0).start()
    idx_cp(0, 0).wait()
    row_cp(0).start()
    idx_cp(1, 1).start()

    def body(cp, carry):
      for b in range(2):
        ci = cp * 2 + b
        row_cp(b).wait()
        idx_cp(ci + 2, b).start()
        idx_cp(ci + 1, 1 - b).wait()
        row_cp(1 - b).start()
        reduce_chunk(ci, b)
      return carry

    lax.fori_loop(0, NCH // 2 - 1, body, 0)

    row_cp(0).wait()
    idx_cp(NCH - 1, 1).wait()
    row_cp(1).start()
    reduce_chunk(NCH - 2, 0)
    row_cp(1).wait()
    reduce_chunk(NCH - 1, 1)

    pltpu.sync_copy(stage_v, out_hbm.at[pl.ds(wid * B_PER_W, B_PER_W)])

  return k(ids_flat, emb)


def _tc_head(pooled, gamma, beta, W, b):
  def body(x_ref, g_ref, be_ref, w_ref, b_ref, o_ref):
    x = x_ref[...] * (1.0 / L)
    mu = jnp.mean(x, axis=-1, keepdims=True)
    xc = x - mu
    var = jnp.mean(xc * xc, axis=-1, keepdims=True)
    y = xc * lax.rsqrt(var + 1e-5) * g_ref[...] + be_ref[...]
    o_ref[...] = (
        lax.dot_general(y, w_ref[...], (((1,), (1,)), ((), ())),
                        preferred_element_type=jnp.float32)
        + b_ref[...]
    )

  return pl.pallas_call(
      body,
      out_shape=jax.ShapeDtypeStruct((B, NUM_CLASSES), jnp.float32),
  )(pooled, gamma.reshape(1, D), beta.reshape(1, D), W,
    b.reshape(1, NUM_CLASSES))


def kernel(input_ids, emb, gamma, beta, W, b):
  ids = input_ids.astype(jnp.int32).reshape(-1)
  pooled = _sc_pool(ids, emb)
  return _tc_head(pooled, gamma, beta, W, b)


# speedup vs baseline: 38.3271x; 1.4889x over previous
import functools

import jax
import jax.numpy as jnp
from jax import lax
from jax.experimental import pallas as pl
from jax.experimental.pallas import tpu as pltpu
from jax.experimental.pallas import tpu_sc as plsc

VOCAB = 50257
D = 64
B = 16384
L = 200
NUM_CLASSES = 2

NC = 2
NS = 16
NW = NC * NS
B_PER_W = B // NW
CHUNK = 4
JPC = CHUNK * L
NCH = B_PER_W // CHUNK
UNROLL = 8

_PERM = (
    [2 * i for i in range(16)] + [2 * i + 1 for i in range(16)]
    + [32 + 2 * i for i in range(16)] + [33 + 2 * i for i in range(16)]
)


def _sc_pool(ids_flat, emb_bf):
  mesh = plsc.VectorSubcoreMesh(core_axis_name="c", subcore_axis_name="s")

  @functools.partial(
      pl.kernel,
      mesh=mesh,
      out_type=jax.ShapeDtypeStruct((B, D), jnp.float32),
      scratch_types=[
          pltpu.VMEM((2, JPC), jnp.int32),
          pltpu.VMEM((2, JPC, D), jnp.bfloat16),
          pltpu.VMEM((B_PER_W, D), jnp.float32),
          pltpu.SemaphoreType.DMA,
          pltpu.SemaphoreType.DMA,
          pltpu.SemaphoreType.DMA,
          pltpu.SemaphoreType.DMA,
      ],
      compiler_params=pltpu.CompilerParams(
          use_tc_tiling_on_sc=False, needs_layout_passes=False),
  )
  def k(ids_hbm, table_hbm, out_hbm, idx_v, rows_v, stage_v,
        sem_i0, sem_i1, sem_r0, sem_r1):
    sem_i = (sem_i0, sem_i1)
    sem_r = (sem_r0, sem_r1)
    wid = lax.axis_index("s") * NC + lax.axis_index("c")
    ibase = wid * (B_PER_W * L)

    def idx_cp(ci, b):
      return pltpu.make_async_copy(
          ids_hbm.at[pl.ds(ibase + ci * JPC, JPC)], idx_v.at[b], sem_i[b])

    def row_cp(b):
      return pltpu.make_async_copy(
          table_hbm.at[idx_v.at[b]], rows_v.at[b], sem_r[b])

    def reduce_chunk(ci, b):
      for r in range(CHUNK):
        def red(jj, accs, r=r, b=b):
          new = list(accs)
          for u in range(UNROLL):
            j = jj * UNROLL + u
            par = u % 2
            for h in range(2):
              v = rows_v[b, r * L + j, pl.ds(h * 32, 32)]
              even, odd = plsc.unpack(v, format=plsc.PackFormat.INTERLEAVED)
              c = par * 4 + h * 2
              new[c] = new[c] + even
              new[c + 1] = new[c + 1] + odd
          return tuple(new)

        accs = lax.fori_loop(
            0, L // UNROLL, red,
            tuple(jnp.zeros((16,), jnp.float32) for _ in range(8)))
        out_row = ci * CHUNK + r
        for q in range(4):
          stage_v[out_row, pl.ds(q * 16, 16)] = accs[q] + accs[4 + q]

    idx_cp(0, 0).start()
    idx_cp(0, 0).wait()
    row_cp(0).start()
    idx_cp(1, 1).start()

    def body(cp, carry):
      for b in range(2):
        ci = cp * 2 + b
        row_cp(b).wait()
        idx_cp(ci + 2, b).start()
        idx_cp(ci + 1, 1 - b).wait()
        row_cp(1 - b).start()
        reduce_chunk(ci, b)
      return carry

    lax.fori_loop(0, NCH // 2 - 1, body, 0)

    row_cp(0).wait()
    idx_cp(NCH - 1, 1).wait()
    row_cp(1).start()
    reduce_chunk(NCH - 2, 0)
    row_cp(1).wait()
    reduce_chunk(NCH - 1, 1)

    pltpu.sync_copy(stage_v, out_hbm.at[pl.ds(wid * B_PER_W, B_PER_W)])

  return k(ids_flat, emb_bf)


def _tc_head(pooled, gamma, beta, W, b):
  def body(x_ref, g_ref, be_ref, w_ref, b_ref, o_ref):
    x = x_ref[...] * (1.0 / L)
    mu = jnp.mean(x, axis=-1, keepdims=True)
    xc = x - mu
    var = jnp.mean(xc * xc, axis=-1, keepdims=True)
    y = xc * lax.rsqrt(var + 1e-5) * g_ref[...] + be_ref[...]
    o_ref[...] = (
        lax.dot_general(y, w_ref[...], (((1,), (1,)), ((), ())),
                        preferred_element_type=jnp.float32)
        + b_ref[...]
    )

  return pl.pallas_call(
      body,
      out_shape=jax.ShapeDtypeStruct((B, NUM_CLASSES), jnp.float32),
  )(pooled, gamma.reshape(1, D), beta.reshape(1, D), W,
    b.reshape(1, NUM_CLASSES))


def kernel(input_ids, emb, gamma, beta, W, b):
  ids = input_ids.astype(jnp.int32).reshape(-1)
  perm = jnp.asarray(_PERM, dtype=jnp.int32)
  pooled = _sc_pool(ids, emb.astype(jnp.bfloat16))
  return _tc_head(pooled, gamma[perm], beta[perm], W[:, perm], b)


# speedup vs baseline: 41.6820x; 1.0875x over previous
import functools

import jax
import jax.numpy as jnp
from jax import lax
from jax.experimental import pallas as pl
from jax.experimental.pallas import tpu as pltpu
from jax.experimental.pallas import tpu_sc as plsc

VOCAB = 50257
D = 64
B = 16384
L = 200
NUM_CLASSES = 2

NC = 2
NS = 16
NW = NC * NS
B_PER_W = B // NW
CHUNK = 4
JPC = CHUNK * L
NCH = B_PER_W // CHUNK
NBUF = 3
UNROLL = 8

_PERM = (
    [2 * i for i in range(16)] + [2 * i + 1 for i in range(16)]
    + [32 + 2 * i for i in range(16)] + [33 + 2 * i for i in range(16)]
)


def _rsqrt_newton(x):
  bits = plsc.bitcast(x, jnp.int32)
  seed = plsc.bitcast(jnp.int32(0x5F3759DF) - (bits >> 1), jnp.float32)
  y = seed
  for _ in range(3):
    y = y * (1.5 - 0.5 * x * y * y)
  return y


def _sc_fused(ids_flat, emb_bf, gamma_p, beta_p, w_p, bias):
  mesh = plsc.VectorSubcoreMesh(core_axis_name="c", subcore_axis_name="s")

  @functools.partial(
      pl.kernel,
      mesh=mesh,
      out_type=jax.ShapeDtypeStruct((B // CHUNK, 16), jnp.float32),
      scratch_types=[
          pltpu.VMEM((NBUF, JPC), jnp.int32),
          pltpu.VMEM((NBUF, JPC, D), jnp.bfloat16),
          pltpu.VMEM((NCH, 16), jnp.float32),
          pltpu.VMEM((D,), jnp.float32),
          pltpu.VMEM((D,), jnp.float32),
          pltpu.VMEM((NUM_CLASSES, D), jnp.float32),
          pltpu.VMEM((16,), jnp.float32),
          pltpu.SemaphoreType.DMA,
          pltpu.SemaphoreType.DMA,
          pltpu.SemaphoreType.DMA,
          pltpu.SemaphoreType.DMA,
          pltpu.SemaphoreType.DMA,
          pltpu.SemaphoreType.DMA,
          pltpu.SemaphoreType.DMA,
      ],
      compiler_params=pltpu.CompilerParams(
          use_tc_tiling_on_sc=False, needs_layout_passes=False),
  )
  def k(ids_hbm, table_hbm, gamma_hbm, beta_hbm, w_hbm, bias_hbm, out_hbm,
        idx_v, rows_v, stage_v, g_v, be_v, w_v, bias_v,
        sem_i0, sem_i1, sem_i2, sem_r0, sem_r1, sem_r2, sem_w):
    sem_i = (sem_i0, sem_i1, sem_i2)
    sem_r = (sem_r0, sem_r1, sem_r2)
    wid = lax.axis_index("s") * NC + lax.axis_index("c")
    ibase = wid * (B_PER_W * L)

    pltpu.async_copy(gamma_hbm, g_v, sem_w).wait()
    pltpu.async_copy(beta_hbm, be_v, sem_w).wait()
    pltpu.async_copy(w_hbm, w_v, sem_w).wait()
    pltpu.async_copy(bias_hbm, bias_v, sem_w).wait()

    def idx_cp(ci, b):
      return pltpu.make_async_copy(
          ids_hbm.at[pl.ds(ibase + ci * JPC, JPC)], idx_v.at[b], sem_i[b])

    def row_cp(b):
      return pltpu.make_async_copy(
          table_hbm.at[idx_v.at[b]], rows_v.at[b], sem_r[b])

    def process_chunk(ci, b):
      lanes = lax.iota(jnp.int32, 16)
      res = jnp.zeros((16,), jnp.float32)
      for r in range(CHUNK):
        def red(jj, accs, r=r, b=b):
          new = list(accs)
          for u in range(UNROLL):
            j = jj * UNROLL + u
            par = u % 2
            for h in range(2):
              v = rows_v[b, r * L + j, pl.ds(h * 32, 32)]
              even, odd = plsc.unpack(v, format=plsc.PackFormat.INTERLEAVED)
              c = par * 4 + h * 2
              new[c] = new[c] + even
              new[c + 1] = new[c + 1] + odd
          return tuple(new)

        accs = lax.fori_loop(
            0, L // UNROLL, red,
            tuple(jnp.zeros((16,), jnp.float32) for _ in range(8)))
        x = [(accs[q] + accs[4 + q]) * (1.0 / L) for q in range(4)]

        mu = jnp.sum((x[0] + x[1]) + (x[2] + x[3])) * (1.0 / D)
        xc = [xq - mu for xq in x]
        sq = (xc[0] * xc[0] + xc[1] * xc[1]) + (xc[2] * xc[2] + xc[3] * xc[3])
        var = jnp.sum(sq) * (1.0 / D)
        rs = _rsqrt_newton(jnp.full((16,), var + 1e-5, jnp.float32))
        y = [
            xc[q] * rs * g_v[pl.ds(16 * q, 16)] + be_v[pl.ds(16 * q, 16)]
            for q in range(4)
        ]

        for cl in range(NUM_CLASSES):
          prods = [y[q] * w_v[cl, pl.ds(16 * q, 16)] for q in range(4)]
          o = jnp.sum((prods[0] + prods[1]) + (prods[2] + prods[3]))
          res = jnp.where(lanes == 2 * r + cl, o, res)
      stage_v[ci, pl.ds(0, 16)] = res + bias_v[pl.ds(0, 16)]

    for t in range(NBUF):
      idx_cp(t, t).start()
    idx_cp(0, 0).wait()
    row_cp(0).start()
    idx_cp(1, 1).wait()
    row_cp(1).start()

    def body(cp, carry):
      for b in range(NBUF):
        ci = cp * NBUF + b
        row_cp(b).wait()

        @pl.when(ci + NBUF < NCH)
        def _():
          idx_cp(ci + NBUF, b).start()

        idx_cp(ci + 2, (b + 2) % 3).wait()
        row_cp((b + 2) % 3).start()
        process_chunk(ci, b)
      return carry

    lax.fori_loop(0, (NCH - 2) // NBUF, body, 0)

    for ci in range(NCH - 2, NCH):
      b = ci % 3
      row_cp(b).wait()
      process_chunk(ci, b)

    pltpu.sync_copy(stage_v, out_hbm.at[pl.ds(wid * NCH, NCH)])

  return k(ids_flat, emb_bf, gamma_p, beta_p, w_p, bias)


def kernel(input_ids, emb, gamma, beta, W, b):
  ids = input_ids.astype(jnp.int32).reshape(-1)
  perm = jnp.asarray(_PERM, dtype=jnp.int32)
  bias_pad = jnp.zeros((16,), jnp.float32).at[: 2 * CHUNK].set(jnp.tile(b, CHUNK))
  packed = _sc_fused(ids, emb.astype(jnp.bfloat16), gamma[perm], beta[perm],
                     W[:, perm], bias_pad)
  return packed[:, : 2 * CHUNK].reshape(B, NUM_CLASSES)
